# initial kernel scaffold (unmeasured)
import jax
import jax.numpy as jnp
from jax import lax
from jax.experimental import pallas as pl
from jax.experimental.pallas import tpu as pltpu

N_DEV = 8


def kernel(x, w_mat):
    m_per, k = x.shape
    _, n_per = w_mat.shape

    x = x.astype(jnp.bfloat16)
    w_mat = w_mat.astype(jnp.bfloat16)

    def body(
        x_ref,
        w_ref,
        out_ref,
        comm_ref,
        amax_ref,
        send_sems,
        recv_sems,
        a_send_sems,
        a_recv_sems,
    ):
        my_pos = lax.axis_index("i")
        left = (my_pos - 1) % N_DEV
        right = (my_pos + 1) % N_DEV

        barrier_sem = pltpu.get_barrier_semaphore()
        for nbr in [left, right]:
            pl.semaphore_signal(
                barrier_sem,
                inc=1,
                device_id=(nbr,),
                device_id_type=pl.DeviceIdType.MESH,
            )
        pl.semaphore_wait(barrier_sem, 2)

        comm_ref[0] = x_ref[...]
        own = jnp.dot(x_ref[...], w_ref[...], preferred_element_type=jnp.float32)
        out_ref[pl.ds(my_pos * m_per, m_per), :] = own
        amax = jnp.max(jnp.abs(own))

        for h in range(N_DEV - 1):
            send_slot = h % 2
            recv_slot = (h + 1) % 2
            rdma = pltpu.make_async_remote_copy(
                src_ref=comm_ref.at[send_slot],
                dst_ref=comm_ref.at[recv_slot],
                send_sem=send_sems.at[send_slot],
                recv_sem=recv_sems.at[recv_slot],
                device_id=(right,),
                device_id_type=pl.DeviceIdType.MESH,
            )
            rdma.start()
            rdma.wait()
            origin = (my_pos - h - 1) % N_DEV
            block = jnp.dot(
                comm_ref[recv_slot], w_ref[...], preferred_element_type=jnp.float32
            )
            out_ref[pl.ds(origin * m_per, m_per), :] = block
            amax = jnp.maximum(amax, jnp.max(jnp.abs(block)))

        amax_ref[my_pos] = jnp.full((8, 128), amax, jnp.float32)
        send_descs = []
        for d in range(1, N_DEV):
            tgt = (my_pos + d) % N_DEV
            s = pltpu.make_async_remote_copy(
                src_ref=amax_ref.at[my_pos],
                dst_ref=amax_ref.at[my_pos],
                send_sem=a_send_sems.at[d],
                recv_sem=a_recv_sems.at[d],
                device_id=(tgt,),
                device_id_type=pl.DeviceIdType.MESH,
            )
            s.start()
            send_descs.append(s)
        for d in range(1, N_DEV):
            src_pos = (my_pos - d) % N_DEV
            r = pltpu.make_async_remote_copy(
                src_ref=amax_ref.at[my_pos],
                dst_ref=amax_ref.at[src_pos],
                send_sem=a_send_sems.at[d],
                recv_sem=a_recv_sems.at[d],
                device_id=(src_pos,),
                device_id_type=pl.DeviceIdType.MESH,
            )
            r.wait_recv()
        for s in send_descs:
            s.wait_send()

        g_amax = jnp.max(amax_ref[...])
        scale = g_amax / 127.0
        q = jnp.clip(jnp.round(out_ref[...] / scale), -127.0, 127.0)
        out_ref[...] = q * scale

    return pl.pallas_call(
        body,
        out_shape=jax.ShapeDtypeStruct((N_DEV * m_per, n_per), jnp.float32),
        in_specs=[
            pl.BlockSpec(memory_space=pltpu.VMEM),
            pl.BlockSpec(memory_space=pltpu.VMEM),
        ],
        out_specs=pl.BlockSpec(memory_space=pltpu.VMEM),
        scratch_shapes=[
            pltpu.VMEM((2, m_per, k), jnp.bfloat16),
            pltpu.VMEM((N_DEV, 8, 128), jnp.float32),
            pltpu.SemaphoreType.DMA((2,)),
            pltpu.SemaphoreType.DMA((2,)),
            pltpu.SemaphoreType.DMA((N_DEV,)),
            pltpu.SemaphoreType.DMA((N_DEV,)),
        ],
        compiler_params=pltpu.CompilerParams(collective_id=0),
    )(x, w_mat)


# baseline (device time: 416443 ns/iter reference)
import jax
import jax.numpy as jnp
from jax import lax
from jax.experimental import pallas as pl
from jax.experimental.pallas import tpu as pltpu

N_DEV = 8


def kernel(x, w_mat):
    m_per, k = x.shape
    _, n_per = w_mat.shape

    x = x.astype(jnp.bfloat16)
    w_mat = w_mat.astype(jnp.bfloat16)

    def body(
        x_ref,
        w_ref,
        out_ref,
        comm_ref,
        amax_ref,
        send_sems,
        recv_sems,
        a_send_sems,
        a_recv_sems,
    ):
        my_pos = lax.axis_index("i")
        left = (my_pos - 1) % N_DEV
        right = (my_pos + 1) % N_DEV

        barrier_sem = pltpu.get_barrier_semaphore()
        for nbr in [left, right]:
            pl.semaphore_signal(
                barrier_sem,
                inc=1,
                device_id=(nbr,),
                device_id_type=pl.DeviceIdType.MESH,
            )
        pl.semaphore_wait(barrier_sem, 2)

        comm_ref[0] = x_ref[...]
        own = jnp.dot(x_ref[...], w_ref[...], preferred_element_type=jnp.float32)
        out_ref[pl.ds(my_pos * m_per, m_per), :] = own
        amax = jnp.max(jnp.abs(own))

        for h in range(N_DEV - 1):
            send_slot = h % 2
            recv_slot = (h + 1) % 2
            rdma = pltpu.make_async_remote_copy(
                src_ref=comm_ref.at[send_slot],
                dst_ref=comm_ref.at[recv_slot],
                send_sem=send_sems.at[send_slot],
                recv_sem=recv_sems.at[recv_slot],
                device_id=(right,),
                device_id_type=pl.DeviceIdType.MESH,
            )
            rdma.start()
            rdma.wait()
            origin = (my_pos - h - 1) % N_DEV
            block = jnp.dot(
                comm_ref[recv_slot], w_ref[...], preferred_element_type=jnp.float32
            )
            out_ref[pl.ds(origin * m_per, m_per), :] = block
            amax = jnp.maximum(amax, jnp.max(jnp.abs(block)))

        amax_ref[my_pos] = jnp.full((8, 128), amax, jnp.float32)
        send_descs = []
        for d in range(1, N_DEV):
            tgt = (my_pos + d) % N_DEV
            s = pltpu.make_async_remote_copy(
                src_ref=amax_ref.at[my_pos],
                dst_ref=amax_ref.at[my_pos],
                send_sem=a_send_sems.at[d],
                recv_sem=a_recv_sems.at[d],
                device_id=(tgt,),
                device_id_type=pl.DeviceIdType.MESH,
            )
            s.start()
            send_descs.append(s)
        for d in range(1, N_DEV):
            src_pos = (my_pos - d) % N_DEV
            r = pltpu.make_async_remote_copy(
                src_ref=amax_ref.at[my_pos],
                dst_ref=amax_ref.at[src_pos],
                send_sem=a_send_sems.at[d],
                recv_sem=a_recv_sems.at[d],
                device_id=(src_pos,),
                device_id_type=pl.DeviceIdType.MESH,
            )
            r.wait_recv()
        for s in send_descs:
            s.wait_send()

        g_amax = jnp.max(amax_ref[...])
        scale = g_amax / 127.0
        q = jnp.clip(jnp.round(out_ref[...] / scale), -127.0, 127.0)
        out_ref[...] = q * scale

    return pl.pallas_call(
        body,
        out_shape=jax.ShapeDtypeStruct((N_DEV * m_per, n_per), jnp.float32),
        in_specs=[
            pl.BlockSpec(memory_space=pltpu.VMEM),
            pl.BlockSpec(memory_space=pltpu.VMEM),
        ],
        out_specs=pl.BlockSpec(memory_space=pltpu.VMEM),
        scratch_shapes=[
            pltpu.VMEM((2, m_per, k), jnp.bfloat16),
            pltpu.VMEM((N_DEV, 8, 128), jnp.float32),
            pltpu.SemaphoreType.DMA((2,)),
            pltpu.SemaphoreType.DMA((2,)),
            pltpu.SemaphoreType.DMA((N_DEV,)),
            pltpu.SemaphoreType.DMA((N_DEV,)),
        ],
        compiler_params=pltpu.CompilerParams(
            collective_id=0, vmem_limit_bytes=100 * 1024 * 1024
        ),
    )(x, w_mat)


# device time: 227490 ns/iter; 1.8306x vs baseline; 1.8306x over previous
import jax
import jax.numpy as jnp
from jax import lax
from jax.experimental import pallas as pl
from jax.experimental.pallas import tpu as pltpu

N_DEV = 8


def kernel(x, w_mat):
    m_per, k = x.shape
    _, n_per = w_mat.shape
    half = m_per // 2

    x = x.astype(jnp.bfloat16)
    w_mat = w_mat.astype(jnp.bfloat16)

    def body(
        x_ref,
        w_ref,
        out_ref,
        cw_ref,
        ccw_ref,
        amax_ref,
        cw_send_sems,
        cw_recv_sems,
        ccw_send_sems,
        ccw_recv_sems,
        a_send_sems,
        a_recv_sems,
    ):
        my_pos = lax.axis_index("i")
        left = (my_pos - 1) % N_DEV
        right = (my_pos + 1) % N_DEV

        barrier_sem = pltpu.get_barrier_semaphore()
        for nbr in [left, right]:
            pl.semaphore_signal(
                barrier_sem,
                inc=1,
                device_id=(nbr,),
                device_id_type=pl.DeviceIdType.MESH,
            )
        pl.semaphore_wait(barrier_sem, 2)

        w = w_ref[...]
        cw_ref[0] = x_ref[pl.ds(0, half), :]
        ccw_ref[0] = x_ref[pl.ds(half, half), :]
        amax = jnp.float32(0.0)

        for h in range(N_DEV - 1):
            ss = h % 2
            rs = (h + 1) % 2
            cw = pltpu.make_async_remote_copy(
                src_ref=cw_ref.at[ss],
                dst_ref=cw_ref.at[rs],
                send_sem=cw_send_sems.at[ss],
                recv_sem=cw_recv_sems.at[rs],
                device_id=(right,),
                device_id_type=pl.DeviceIdType.MESH,
            )
            ccw = pltpu.make_async_remote_copy(
                src_ref=ccw_ref.at[ss],
                dst_ref=ccw_ref.at[rs],
                send_sem=ccw_send_sems.at[ss],
                recv_sem=ccw_recv_sems.at[rs],
                device_id=(left,),
                device_id_type=pl.DeviceIdType.MESH,
            )
            cw.start()
            ccw.start()

            if h == 0:
                own = jnp.dot(x_ref[...], w, preferred_element_type=jnp.float32)
                out_ref[pl.ds(my_pos * m_per, m_per), :] = own
                amax = jnp.maximum(amax, jnp.max(jnp.abs(own)))
            else:
                o_cw = (my_pos - h) % N_DEV
                blk = jnp.dot(cw_ref[ss], w, preferred_element_type=jnp.float32)
                out_ref[pl.ds(o_cw * m_per, half), :] = blk
                amax = jnp.maximum(amax, jnp.max(jnp.abs(blk)))
                o_ccw = (my_pos + h) % N_DEV
                blk = jnp.dot(ccw_ref[ss], w, preferred_element_type=jnp.float32)
                out_ref[pl.ds(o_ccw * m_per + half, half), :] = blk
                amax = jnp.maximum(amax, jnp.max(jnp.abs(blk)))

            cw.wait()
            ccw.wait()

        ls = (N_DEV - 1) % 2
        o_cw = right
        blk = jnp.dot(cw_ref[ls], w, preferred_element_type=jnp.float32)
        out_ref[pl.ds(o_cw * m_per, half), :] = blk
        amax = jnp.maximum(amax, jnp.max(jnp.abs(blk)))
        o_ccw = left
        blk = jnp.dot(ccw_ref[ls], w, preferred_element_type=jnp.float32)
        out_ref[pl.ds(o_ccw * m_per + half, half), :] = blk
        amax = jnp.maximum(amax, jnp.max(jnp.abs(blk)))

        amax_ref[my_pos] = jnp.full((8, 128), amax, jnp.float32)
        send_descs = []
        for d in range(1, N_DEV):
            tgt = (my_pos + d) % N_DEV
            s = pltpu.make_async_remote_copy(
                src_ref=amax_ref.at[my_pos],
                dst_ref=amax_ref.at[my_pos],
                send_sem=a_send_sems.at[d],
                recv_sem=a_recv_sems.at[d],
                device_id=(tgt,),
                device_id_type=pl.DeviceIdType.MESH,
            )
            s.start()
            send_descs.append(s)
        for d in range(1, N_DEV):
            src_pos = (my_pos - d) % N_DEV
            r = pltpu.make_async_remote_copy(
                src_ref=amax_ref.at[my_pos],
                dst_ref=amax_ref.at[src_pos],
                send_sem=a_send_sems.at[d],
                recv_sem=a_recv_sems.at[d],
                device_id=(src_pos,),
                device_id_type=pl.DeviceIdType.MESH,
            )
            r.wait_recv()
        for s in send_descs:
            s.wait_send()

        g_amax = jnp.max(amax_ref[...])
        scale = g_amax / 127.0
        q = jnp.clip(jnp.round(out_ref[...] / scale), -127.0, 127.0)
        out_ref[...] = q * scale

    return pl.pallas_call(
        body,
        out_shape=jax.ShapeDtypeStruct((N_DEV * m_per, n_per), jnp.float32),
        in_specs=[
            pl.BlockSpec(memory_space=pltpu.VMEM),
            pl.BlockSpec(memory_space=pltpu.VMEM),
        ],
        out_specs=pl.BlockSpec(memory_space=pltpu.VMEM),
        scratch_shapes=[
            pltpu.VMEM((2, half, k), jnp.bfloat16),
            pltpu.VMEM((2, half, k), jnp.bfloat16),
            pltpu.VMEM((N_DEV, 8, 128), jnp.float32),
            pltpu.SemaphoreType.DMA((2,)),
            pltpu.SemaphoreType.DMA((2,)),
            pltpu.SemaphoreType.DMA((2,)),
            pltpu.SemaphoreType.DMA((2,)),
            pltpu.SemaphoreType.DMA((N_DEV,)),
            pltpu.SemaphoreType.DMA((N_DEV,)),
        ],
        compiler_params=pltpu.CompilerParams(
            collective_id=0, vmem_limit_bytes=100 * 1024 * 1024
        ),
    )(x, w_mat)


# device time: 215748 ns/iter; 1.9302x vs baseline; 1.0544x over previous
import jax
import jax.numpy as jnp
from jax import lax
from jax.experimental import pallas as pl
from jax.experimental.pallas import tpu as pltpu

N_DEV = 8


def kernel(x, w_mat):
    m_per, k = x.shape
    _, n_per = w_mat.shape
    half = m_per // 2

    def body(
        x_ref,
        w_ref,
        out_ref,
        w_bf_ref,
        cw_ref,
        ccw_ref,
        amax_ref,
        cw_send_sems,
        cw_recv_sems,
        ccw_send_sems,
        ccw_recv_sems,
        a_send_sems,
        a_recv_sems,
    ):
        my_pos = lax.axis_index("i")
        left = (my_pos - 1) % N_DEV
        right = (my_pos + 1) % N_DEV

        barrier_sem = pltpu.get_barrier_semaphore()
        for nbr in [left, right]:
            pl.semaphore_signal(
                barrier_sem,
                inc=1,
                device_id=(nbr,),
                device_id_type=pl.DeviceIdType.MESH,
            )
        pl.semaphore_wait(barrier_sem, 2)

        cw_ref[0] = x_ref[pl.ds(0, half), :].astype(jnp.bfloat16)
        ccw_ref[0] = x_ref[pl.ds(half, half), :].astype(jnp.bfloat16)
        amax = jnp.float32(0.0)

        for h in range(N_DEV - 1):
            ss = h % 2
            rs = (h + 1) % 2
            cw = pltpu.make_async_remote_copy(
                src_ref=cw_ref.at[ss],
                dst_ref=cw_ref.at[rs],
                send_sem=cw_send_sems.at[ss],
                recv_sem=cw_recv_sems.at[rs],
                device_id=(right,),
                device_id_type=pl.DeviceIdType.MESH,
            )
            ccw = pltpu.make_async_remote_copy(
                src_ref=ccw_ref.at[ss],
                dst_ref=ccw_ref.at[rs],
                send_sem=ccw_send_sems.at[ss],
                recv_sem=ccw_recv_sems.at[rs],
                device_id=(left,),
                device_id_type=pl.DeviceIdType.MESH,
            )
            cw.start()
            ccw.start()

            if h == 0:
                w_bf_ref[...] = w_ref[...].astype(jnp.bfloat16)
                w = w_bf_ref[...]
                o_cw, o_ccw = my_pos, my_pos
            else:
                o_cw = (my_pos - h) % N_DEV
                o_ccw = (my_pos + h) % N_DEV
            blk = jnp.dot(cw_ref[ss], w, preferred_element_type=jnp.float32)
            out_ref[pl.ds(o_cw * m_per, half), :] = blk
            amax = jnp.maximum(amax, jnp.max(jnp.abs(blk)))
            blk = jnp.dot(ccw_ref[ss], w, preferred_element_type=jnp.float32)
            out_ref[pl.ds(o_ccw * m_per + half, half), :] = blk
            amax = jnp.maximum(amax, jnp.max(jnp.abs(blk)))

            cw.wait()
            ccw.wait()

        ls = (N_DEV - 1) % 2
        o_cw = right
        blk = jnp.dot(cw_ref[ls], w, preferred_element_type=jnp.float32)
        out_ref[pl.ds(o_cw * m_per, half), :] = blk
        amax = jnp.maximum(amax, jnp.max(jnp.abs(blk)))
        o_ccw = left
        blk = jnp.dot(ccw_ref[ls], w, preferred_element_type=jnp.float32)
        out_ref[pl.ds(o_ccw * m_per + half, half), :] = blk
        amax = jnp.maximum(amax, jnp.max(jnp.abs(blk)))

        amax_ref[my_pos] = jnp.full((8, 128), amax, jnp.float32)
        send_descs = []
        for d in range(1, N_DEV):
            tgt = (my_pos + d) % N_DEV
            s = pltpu.make_async_remote_copy(
                src_ref=amax_ref.at[my_pos],
                dst_ref=amax_ref.at[my_pos],
                send_sem=a_send_sems.at[d],
                recv_sem=a_recv_sems.at[d],
                device_id=(tgt,),
                device_id_type=pl.DeviceIdType.MESH,
            )
            s.start()
            send_descs.append(s)
        for d in range(1, N_DEV):
            src_pos = (my_pos - d) % N_DEV
            r = pltpu.make_async_remote_copy(
                src_ref=amax_ref.at[my_pos],
                dst_ref=amax_ref.at[src_pos],
                send_sem=a_send_sems.at[d],
                recv_sem=a_recv_sems.at[d],
                device_id=(src_pos,),
                device_id_type=pl.DeviceIdType.MESH,
            )
            r.wait_recv()
        for s in send_descs:
            s.wait_send()

        g_amax = jnp.max(amax_ref[...])
        inv = 127.0 / g_amax
        scale = g_amax * (1.0 / 127.0)
        q = jnp.clip(jnp.round(out_ref[...] * inv), -127.0, 127.0)
        out_ref[...] = q * scale

    return pl.pallas_call(
        body,
        out_shape=jax.ShapeDtypeStruct((N_DEV * m_per, n_per), jnp.float32),
        in_specs=[
            pl.BlockSpec(memory_space=pltpu.VMEM),
            pl.BlockSpec(memory_space=pltpu.VMEM),
        ],
        out_specs=pl.BlockSpec(memory_space=pltpu.VMEM),
        scratch_shapes=[
            pltpu.VMEM((k, n_per), jnp.bfloat16),
            pltpu.VMEM((2, half, k), jnp.bfloat16),
            pltpu.VMEM((2, half, k), jnp.bfloat16),
            pltpu.VMEM((N_DEV, 8, 128), jnp.float32),
            pltpu.SemaphoreType.DMA((2,)),
            pltpu.SemaphoreType.DMA((2,)),
            pltpu.SemaphoreType.DMA((2,)),
            pltpu.SemaphoreType.DMA((2,)),
            pltpu.SemaphoreType.DMA((N_DEV,)),
            pltpu.SemaphoreType.DMA((N_DEV,)),
        ],
        compiler_params=pltpu.CompilerParams(
            collective_id=0, vmem_limit_bytes=100 * 1024 * 1024
        ),
    )(x, w_mat)


# device time: 202359 ns/iter; 2.0579x vs baseline; 1.0662x over previous
import jax
import jax.numpy as jnp
from jax import lax
from jax.experimental import pallas as pl
from jax.experimental.pallas import tpu as pltpu

N_DEV = 8
N_FLOW = 4


def kernel(x, w_mat):
    m_per, k = x.shape
    _, n_per = w_mat.shape
    q_rows = m_per // 4

    def body(
        x_ref,
        w_ref,
        out_ref,
        w_bf_ref,
        flow_ref,
        amax_ref,
        send_sems,
        recv_sems,
        a_send_sems,
        a_recv_sems,
    ):
        my_pos = lax.axis_index("i")
        left = (my_pos - 1) % N_DEV
        right = (my_pos + 1) % N_DEV

        barrier_sem = pltpu.get_barrier_semaphore()
        for nbr in [left, right]:
            pl.semaphore_signal(
                barrier_sem,
                inc=1,
                device_id=(nbr,),
                device_id_type=pl.DeviceIdType.MESH,
            )
        pl.semaphore_wait(barrier_sem, 2)

        flow_dst = [right, right, left, left]

        def make_rdma(f, h):
            ss, rs = h % 2, (h + 1) % 2
            return pltpu.make_async_remote_copy(
                src_ref=flow_ref.at[f, ss],
                dst_ref=flow_ref.at[f, rs],
                send_sem=send_sems.at[f, ss],
                recv_sem=recv_sems.at[f, rs],
                device_id=(flow_dst[f],),
                device_id_type=pl.DeviceIdType.MESH,
            )

        def origin(f, h):
            return (my_pos - h - 1) % N_DEV if f < 2 else (my_pos + h + 1) % N_DEV

        flow_order = (0, 2, 1, 3)

        rdmas = [None] * N_FLOW
        for f in flow_order:
            flow_ref[f, 0] = x_ref[pl.ds(f * q_rows, q_rows), :].astype(jnp.bfloat16)
            rdmas[f] = make_rdma(f, 0)
            rdmas[f].start()

        w_bf_ref[...] = w_ref[...].astype(jnp.bfloat16)
        w = w_bf_ref[...]
        amax = jnp.float32(0.0)
        for f in range(N_FLOW):
            blk = jnp.dot(flow_ref[f, 0], w, preferred_element_type=jnp.float32)
            out_ref[pl.ds(my_pos * m_per + f * q_rows, q_rows), :] = blk
            amax = jnp.maximum(amax, jnp.max(jnp.abs(blk)))

        for h in range(N_DEV - 1):
            for f in flow_order:
                rdmas[f].wait()
                if h < N_DEV - 2:
                    rdmas[f] = make_rdma(f, h + 1)
                    rdmas[f].start()
            rs = (h + 1) % 2
            for f in range(N_FLOW):
                o = origin(f, h)
                blk = jnp.dot(flow_ref[f, rs], w, preferred_element_type=jnp.float32)
                row = o * m_per + f * q_rows
                out_ref[pl.ds(row, q_rows), :] = blk
                amax = jnp.maximum(amax, jnp.max(jnp.abs(blk)))

        amax_ref[my_pos] = jnp.full((8, 128), amax, jnp.float32)
        send_descs = []
        for d in range(1, N_DEV):
            tgt = (my_pos + d) % N_DEV
            s = pltpu.make_async_remote_copy(
                src_ref=amax_ref.at[my_pos],
                dst_ref=amax_ref.at[my_pos],
                send_sem=a_send_sems.at[d],
                recv_sem=a_recv_sems.at[d],
                device_id=(tgt,),
                device_id_type=pl.DeviceIdType.MESH,
            )
            s.start()
            send_descs.append(s)
        for d in range(1, N_DEV):
            src_pos = (my_pos - d) % N_DEV
            r = pltpu.make_async_remote_copy(
                src_ref=amax_ref.at[my_pos],
                dst_ref=amax_ref.at[src_pos],
                send_sem=a_send_sems.at[d],
                recv_sem=a_recv_sems.at[d],
                device_id=(src_pos,),
                device_id_type=pl.DeviceIdType.MESH,
            )
            r.wait_recv()
        for s in send_descs:
            s.wait_send()

        g_amax = jnp.max(amax_ref[...])
        inv = 127.0 / g_amax
        scale = g_amax * (1.0 / 127.0)
        q = jnp.clip(jnp.round(out_ref[...] * inv), -127.0, 127.0)
        out_ref[...] = q * scale

    return pl.pallas_call(
        body,
        out_shape=jax.ShapeDtypeStruct((N_DEV * m_per, n_per), jnp.float32),
        in_specs=[
            pl.BlockSpec(memory_space=pltpu.VMEM),
            pl.BlockSpec(memory_space=pltpu.VMEM),
        ],
        out_specs=pl.BlockSpec(memory_space=pltpu.VMEM),
        scratch_shapes=[
            pltpu.VMEM((k, n_per), jnp.bfloat16),
            pltpu.VMEM((N_FLOW, 2, q_rows, k), jnp.bfloat16),
            pltpu.VMEM((N_DEV, 8, 128), jnp.float32),
            pltpu.SemaphoreType.DMA((N_FLOW, 2)),
            pltpu.SemaphoreType.DMA((N_FLOW, 2)),
            pltpu.SemaphoreType.DMA((N_DEV,)),
            pltpu.SemaphoreType.DMA((N_DEV,)),
        ],
        compiler_params=pltpu.CompilerParams(
            collective_id=0, vmem_limit_bytes=100 * 1024 * 1024
        ),
    )(x, w_mat)


# device time: 198364 ns/iter; 2.0994x vs baseline; 1.0201x over previous
import jax
import jax.numpy as jnp
from jax import lax
from jax.experimental import pallas as pl
from jax.experimental.pallas import tpu as pltpu

N_DEV = 8
N_FLOW = 4


def kernel(x, w_mat):
    m_per, k = x.shape
    _, n_per = w_mat.shape
    q_rows = m_per // 4

    def body(
        x_ref,
        w_ref,
        out_ref,
        w_bf_ref,
        flow_ref,
        amax_ref,
        send_sems,
        recv_sems,
        a_send_sems,
        a_recv_sems,
    ):
        my_pos = lax.axis_index("i")
        left = (my_pos - 1) % N_DEV
        right = (my_pos + 1) % N_DEV

        barrier_sem = pltpu.get_barrier_semaphore()
        for nbr in [left, right]:
            pl.semaphore_signal(
                barrier_sem,
                inc=1,
                device_id=(nbr,),
                device_id_type=pl.DeviceIdType.MESH,
            )
        pl.semaphore_wait(barrier_sem, 2)

        flow_dst = [right, right, left, left]

        def make_rdma(f, h):
            ss, rs = h % 2, (h + 1) % 2
            return pltpu.make_async_remote_copy(
                src_ref=flow_ref.at[f, ss],
                dst_ref=flow_ref.at[f, rs],
                send_sem=send_sems.at[f, ss],
                recv_sem=recv_sems.at[f, rs],
                device_id=(flow_dst[f],),
                device_id_type=pl.DeviceIdType.MESH,
            )

        def origin(f, h):
            return (my_pos - h - 1) % N_DEV if f < 2 else (my_pos + h + 1) % N_DEV

        flow_order = (0, 2, 1, 3)

        rdmas = [None] * N_FLOW
        for f in flow_order:
            flow_ref[f, 0] = x_ref[pl.ds(f * q_rows, q_rows), :].astype(jnp.bfloat16)
            rdmas[f] = make_rdma(f, 0)
            rdmas[f].start()

        w_bf_ref[...] = w_ref[...].astype(jnp.bfloat16)
        w = w_bf_ref[...]
        amax = jnp.float32(0.0)
        for f in range(N_FLOW):
            blk = jnp.dot(flow_ref[f, 0], w, preferred_element_type=jnp.float32)
            out_ref[pl.ds(my_pos * m_per + f * q_rows, q_rows), :] = blk
            amax = jnp.maximum(amax, jnp.max(jnp.abs(blk)))

        for h in range(N_DEV - 1):
            for f in flow_order:
                rdmas[f].wait()
                if h < N_DEV - 2:
                    rdmas[f] = make_rdma(f, h + 1)
                    rdmas[f].start()
            rs = (h + 1) % 2
            for f in range(N_FLOW):
                o = origin(f, h)
                blk = jnp.dot(flow_ref[f, rs], w, preferred_element_type=jnp.float32)
                row = o * m_per + f * q_rows
                out_ref[pl.ds(row, q_rows), :] = blk
                amax = jnp.maximum(amax, jnp.max(jnp.abs(blk)))

        amax_ref[my_pos] = jnp.full((8, 128), amax, jnp.float32)
        send_descs = []
        for d in range(1, N_DEV):
            tgt = (my_pos + d) % N_DEV
            s = pltpu.make_async_remote_copy(
                src_ref=amax_ref.at[my_pos],
                dst_ref=amax_ref.at[my_pos],
                send_sem=a_send_sems.at[d],
                recv_sem=a_recv_sems.at[d],
                device_id=(tgt,),
                device_id_type=pl.DeviceIdType.MESH,
            )
            s.start()
            send_descs.append(s)
        for d in range(1, N_DEV):
            src_pos = (my_pos - d) % N_DEV
            r = pltpu.make_async_remote_copy(
                src_ref=amax_ref.at[my_pos],
                dst_ref=amax_ref.at[src_pos],
                send_sem=a_send_sems.at[d],
                recv_sem=a_recv_sems.at[d],
                device_id=(src_pos,),
                device_id_type=pl.DeviceIdType.MESH,
            )
            r.wait_recv()
        for s in send_descs:
            s.wait_send()

        g_amax = jnp.max(amax_ref[...])
        out_ref[0:8, 0:128] = jnp.full((8, 128), g_amax, jnp.float32)

    return pl.pallas_call(
        body,
        out_shape=jax.ShapeDtypeStruct((N_DEV * m_per, n_per), jnp.float32),
        in_specs=[
            pl.BlockSpec(memory_space=pltpu.VMEM),
            pl.BlockSpec(memory_space=pltpu.VMEM),
        ],
        out_specs=pl.BlockSpec(memory_space=pltpu.VMEM),
        scratch_shapes=[
            pltpu.VMEM((k, n_per), jnp.bfloat16),
            pltpu.VMEM((N_FLOW, 2, q_rows, k), jnp.bfloat16),
            pltpu.VMEM((N_DEV, 8, 128), jnp.float32),
            pltpu.SemaphoreType.DMA((N_FLOW, 2)),
            pltpu.SemaphoreType.DMA((N_FLOW, 2)),
            pltpu.SemaphoreType.DMA((N_DEV,)),
            pltpu.SemaphoreType.DMA((N_DEV,)),
        ],
        compiler_params=pltpu.CompilerParams(
            collective_id=0, vmem_limit_bytes=100 * 1024 * 1024
        ),
    )(x, w_mat)


# device time: 155843 ns/iter; 2.6722x vs baseline; 1.2728x over previous
import jax
import jax.numpy as jnp
from jax import lax
from jax.experimental import pallas as pl
from jax.experimental.pallas import tpu as pltpu

N_DEV = 8
N_FLOW = 4
N_HOP = 5

RING = [0, 4, 7, 3, 2, 6, 5, 1]
RIDX = {p: i for i, p in enumerate(RING)}
CHORD = {
    p: RING[(RIDX[p] + (3 if RIDX[p] % 2 == 0 else -3)) % 8] for p in range(8)
}

FLOWPAIR_EVEN = [(2, 3), (2, 3), (0, 1), (0, 1)]
FLOWPAIR_ODD = [(0, 1), (0, 1), (2, 3), (2, 3)]
CHORD_ROFF_EVEN = [2, 2, 1, 1, -2, -2, -1, -1]
CHORD_Q_EVEN = [0, 1, 0, 1, 2, 3, 2, 3]


def _sel(p, table):
    v = jnp.int32(table[0])
    for j in range(1, 8):
        v = jnp.where(p == j, jnp.int32(table[j]), v)
    return v


def _ring_tab(off):
    return [RING[(RIDX[p] + off) % 8] for p in range(8)]


def kernel(x, w_mat):
    m_per, k = x.shape
    _, n_per = w_mat.shape
    q_rows = m_per // 4

    def body(
        x_ref,
        w_ref,
        out_ref,
        w_bf_ref,
        w_stage_ref,
        flow_ref,
        chord_ref,
        amax_ref,
        send_sems,
        recv_sems,
        c_send_sems,
        c_recv_sems,
        a_send_sems,
        a_recv_sems,
        w_dma_sem,
    ):
        p = lax.axis_index("i")
        right = _sel(p, _ring_tab(1))
        left = _sel(p, _ring_tab(-1))
        chordp = _sel(p, [CHORD[q] for q in range(8)])
        parity = _sel(p, [RIDX[q] % 2 for q in range(8)])

        barrier_sem = pltpu.get_barrier_semaphore()
        for nbr in [left, right, chordp]:
            pl.semaphore_signal(
                barrier_sem,
                inc=1,
                device_id=(nbr,),
                device_id_type=pl.DeviceIdType.MESH,
            )
        pl.semaphore_wait(barrier_sem, 3)

        flow_dst = [right, right, left, left]

        def make_rdma(f, h):
            ss, rs = h % 2, (h + 1) % 2
            return pltpu.make_async_remote_copy(
                src_ref=flow_ref.at[f, ss],
                dst_ref=flow_ref.at[f, rs],
                send_sem=send_sems.at[f, ss],
                recv_sem=recv_sems.at[f, rs],
                device_id=(flow_dst[f],),
                device_id_type=pl.DeviceIdType.MESH,
            )

        def chord_desc(src_f, src_slot, t):
            return pltpu.make_async_remote_copy(
                src_ref=flow_ref.at[src_f, src_slot],
                dst_ref=chord_ref.at[t],
                send_sem=c_send_sems.at[t],
                recv_sem=c_recv_sems.at[t],
                device_id=(chordp,),
                device_id_type=pl.DeviceIdType.MESH,
            )

        flow_order = (0, 2, 1, 3)

        rdmas = [None] * N_FLOW
        for f in flow_order:
            flow_ref[f, 0] = x_ref[pl.ds(f * q_rows, q_rows), :].astype(jnp.bfloat16)
            rdmas[f] = make_rdma(f, 0)
            rdmas[f].start()

        kq = k // 4
        for i in range(4):
            cp = pltpu.make_async_copy(
                w_ref.at[pl.ds(i * kq, kq), :], w_stage_ref, w_dma_sem
            )
            cp.start()
            cp.wait()
            w_bf_ref[pl.ds(i * kq, kq), :] = w_stage_ref[...].astype(jnp.bfloat16)
        w = w_bf_ref[...]
        amax = jnp.float32(0.0)
        for f in range(N_FLOW):
            blk = jnp.dot(flow_ref[f, 0], w, preferred_element_type=jnp.float32)
            out_ref[pl.ds(p * m_per + f * q_rows, q_rows), :] = blk
            amax = jnp.maximum(amax, jnp.max(jnp.abs(blk)))

        for h in range(N_HOP):
            rs = (h + 1) % 2
            for f in flow_order:
                rdmas[f].wait()
                if h < N_HOP - 1:
                    rdmas[f] = make_rdma(f, h + 1)
                    rdmas[f].start()

            if h < 4:

                @pl.when(parity == 0)
                def _():
                    for j, f in enumerate(FLOWPAIR_EVEN[h]):
                        chord_desc(f, rs, 2 * h + j).start()

                @pl.when(parity == 1)
                def _():
                    for j, f in enumerate(FLOWPAIR_ODD[h]):
                        chord_desc(f, rs, 2 * h + j).start()

            for f in range(N_FLOW):
                off = -(h + 1) if f < 2 else (h + 1)
                o = _sel(p, _ring_tab(off))
                blk = jnp.dot(flow_ref[f, rs], w, preferred_element_type=jnp.float32)
                out_ref[pl.ds(o * m_per + f * q_rows, q_rows), :] = blk
                amax = jnp.maximum(amax, jnp.max(jnp.abs(blk)))

            if h >= 1:
                for t in (2 * (h - 1), 2 * h - 1):
                    chord_desc(0, 0, t).wait_recv()
                    roff_e = CHORD_ROFF_EVEN[t]
                    o = jnp.where(
                        parity == 0,
                        _sel(p, _ring_tab(roff_e)),
                        _sel(p, _ring_tab(-roff_e)),
                    )
                    q_e = CHORD_Q_EVEN[t]
                    qq = jnp.where(parity == 0, q_e, (q_e + 2) % 4)
                    blk = jnp.dot(
                        chord_ref[t], w, preferred_element_type=jnp.float32
                    )
                    out_ref[pl.ds(o * m_per + qq * q_rows, q_rows), :] = blk
                    amax = jnp.maximum(amax, jnp.max(jnp.abs(blk)))

        for t in range(8):
            chord_desc(0, 0, t).wait_send()

        amax_ref[p] = jnp.full((8, 128), amax, jnp.float32)
        send_descs = []
        for d in range(1, N_DEV):
            tgt = (p + d) % N_DEV
            s = pltpu.make_async_remote_copy(
                src_ref=amax_ref.at[p],
                dst_ref=amax_ref.at[p],
                send_sem=a_send_sems.at[d],
                recv_sem=a_recv_sems.at[d],
                device_id=(tgt,),
                device_id_type=pl.DeviceIdType.MESH,
            )
            s.start()
            send_descs.append(s)
        for d in range(1, N_DEV):
            src_pos = (p - d) % N_DEV
            r = pltpu.make_async_remote_copy(
                src_ref=amax_ref.at[p],
                dst_ref=amax_ref.at[src_pos],
                send_sem=a_send_sems.at[d],
                recv_sem=a_recv_sems.at[d],
                device_id=(src_pos,),
                device_id_type=pl.DeviceIdType.MESH,
            )
            r.wait_recv()
        for s in send_descs:
            s.wait_send()

        g_amax = jnp.max(amax_ref[...])
        inv = 127.0 / g_amax
        scale = g_amax * (1.0 / 127.0)
        for rblk in range(N_DEV):
            sl = pl.ds(rblk * m_per, m_per)
            y = out_ref[sl, :]
            qv = jnp.clip(jnp.round(y * inv), -127.0, 127.0)
            out_ref[sl, :] = qv * scale

    return pl.pallas_call(
        body,
        out_shape=jax.ShapeDtypeStruct((N_DEV * m_per, n_per), jnp.float32),
        in_specs=[
            pl.BlockSpec(memory_space=pltpu.VMEM),
            pl.BlockSpec(memory_space=pl.ANY),
        ],
        out_specs=pl.BlockSpec(memory_space=pltpu.VMEM),
        scratch_shapes=[
            pltpu.VMEM((k, n_per), jnp.bfloat16),
            pltpu.VMEM((k // 4, n_per), jnp.float32),
            pltpu.VMEM((N_FLOW, 2, q_rows, k), jnp.bfloat16),
            pltpu.VMEM((8, q_rows, k), jnp.bfloat16),
            pltpu.VMEM((N_DEV, 8, 128), jnp.float32),
            pltpu.SemaphoreType.DMA((N_FLOW, 2)),
            pltpu.SemaphoreType.DMA((N_FLOW, 2)),
            pltpu.SemaphoreType.DMA((8,)),
            pltpu.SemaphoreType.DMA((8,)),
            pltpu.SemaphoreType.DMA((N_DEV,)),
            pltpu.SemaphoreType.DMA((N_DEV,)),
            pltpu.SemaphoreType.DMA,
        ],
        compiler_params=pltpu.CompilerParams(
            collective_id=0, vmem_limit_bytes=100 * 1024 * 1024
        ),
    )(x, w_mat)


# device time: 155760 ns/iter; 2.6736x vs baseline; 1.0005x over previous
import jax
import jax.numpy as jnp
from jax import lax
from jax.experimental import pallas as pl
from jax.experimental.pallas import tpu as pltpu

N_DEV = 8
N_FLOW = 4
N_HOP = 5

RING = [0, 4, 7, 3, 2, 6, 5, 1]
RIDX = {p: i for i, p in enumerate(RING)}
CHORD = {
    p: RING[(RIDX[p] + (3 if RIDX[p] % 2 == 0 else -3)) % 8] for p in range(8)
}

FLOWPAIR_EVEN = [(2, 3), (2, 3), (0, 1), (0, 1)]
FLOWPAIR_ODD = [(0, 1), (0, 1), (2, 3), (2, 3)]
CHORD_ROFF_EVEN = [2, 2, 1, 1, -2, -2, -1, -1]
CHORD_Q_EVEN = [0, 1, 0, 1, 2, 3, 2, 3]


def _sel(p, table):
    v = jnp.int32(table[0])
    for j in range(1, 8):
        v = jnp.where(p == j, jnp.int32(table[j]), v)
    return v


def _ring_tab(off):
    return [RING[(RIDX[p] + off) % 8] for p in range(8)]


def kernel(x, w_mat):
    m_per, k = x.shape
    _, n_per = w_mat.shape
    q_rows = m_per // 4

    def body(
        x_ref,
        w_ref,
        out_ref,
        w_bf_ref,
        w_stage_ref,
        flow_ref,
        chord_ref,
        amax_ref,
        send_sems,
        recv_sems,
        c_send_sems,
        c_recv_sems,
        a_send_sems,
        a_recv_sems,
        w_dma_sem,
    ):
        p = lax.axis_index("i")
        right = _sel(p, _ring_tab(1))
        left = _sel(p, _ring_tab(-1))
        chordp = _sel(p, [CHORD[q] for q in range(8)])
        parity = _sel(p, [RIDX[q] % 2 for q in range(8)])

        barrier_sem = pltpu.get_barrier_semaphore()
        for nbr in [left, right, chordp]:
            pl.semaphore_signal(
                barrier_sem,
                inc=1,
                device_id=(nbr,),
                device_id_type=pl.DeviceIdType.MESH,
            )
        pl.semaphore_wait(barrier_sem, 3)

        flow_dst = [right, right, left, left]

        def make_rdma(f, h):
            ss, rs = h % 2, (h + 1) % 2
            return pltpu.make_async_remote_copy(
                src_ref=flow_ref.at[f, ss],
                dst_ref=flow_ref.at[f, rs],
                send_sem=send_sems.at[f, ss],
                recv_sem=recv_sems.at[f, rs],
                device_id=(flow_dst[f],),
                device_id_type=pl.DeviceIdType.MESH,
            )

        def chord_desc(src_f, src_slot, t):
            return pltpu.make_async_remote_copy(
                src_ref=flow_ref.at[src_f, src_slot],
                dst_ref=chord_ref.at[t],
                send_sem=c_send_sems.at[t],
                recv_sem=c_recv_sems.at[t],
                device_id=(chordp,),
                device_id_type=pl.DeviceIdType.MESH,
            )

        flow_order = (0, 2, 1, 3)

        rdmas = [None] * N_FLOW
        for f in flow_order:
            flow_ref[f, 0] = x_ref[pl.ds(f * q_rows, q_rows), :].astype(jnp.bfloat16)
            rdmas[f] = make_rdma(f, 0)
            rdmas[f].start()

        kq = k // 4
        for i in range(4):
            cp = pltpu.make_async_copy(
                w_ref.at[pl.ds(i * kq, kq), :], w_stage_ref, w_dma_sem
            )
            cp.start()
            cp.wait()
            w_bf_ref[pl.ds(i * kq, kq), :] = w_stage_ref[...].astype(jnp.bfloat16)
        w = w_bf_ref[...]
        amax = jnp.float32(0.0)

        for h in range(N_HOP):
            rs = (h + 1) % 2
            for f in flow_order:
                rdmas[f].wait()
                if h < N_HOP - 1:
                    rdmas[f] = make_rdma(f, h + 1)
                    rdmas[f].start()

            if h < 4:

                @pl.when(parity == 0)
                def _():
                    for j, f in enumerate(FLOWPAIR_EVEN[h]):
                        chord_desc(f, rs, 2 * h + j).start()

                @pl.when(parity == 1)
                def _():
                    for j, f in enumerate(FLOWPAIR_ODD[h]):
                        chord_desc(f, rs, 2 * h + j).start()

            if h == 0:
                for f in range(N_FLOW):
                    xq = x_ref[pl.ds(f * q_rows, q_rows), :].astype(jnp.bfloat16)
                    blk = jnp.dot(xq, w, preferred_element_type=jnp.float32)
                    out_ref[pl.ds(p * m_per + f * q_rows, q_rows), :] = blk
                    amax = jnp.maximum(amax, jnp.max(jnp.abs(blk)))

            for f in range(N_FLOW):
                off = -(h + 1) if f < 2 else (h + 1)
                o = _sel(p, _ring_tab(off))
                blk = jnp.dot(flow_ref[f, rs], w, preferred_element_type=jnp.float32)
                out_ref[pl.ds(o * m_per + f * q_rows, q_rows), :] = blk
                amax = jnp.maximum(amax, jnp.max(jnp.abs(blk)))

            if h >= 1:
                for t in (2 * (h - 1), 2 * h - 1):
                    chord_desc(0, 0, t).wait_recv()
                    roff_e = CHORD_ROFF_EVEN[t]
                    o = jnp.where(
                        parity == 0,
                        _sel(p, _ring_tab(roff_e)),
                        _sel(p, _ring_tab(-roff_e)),
                    )
                    q_e = CHORD_Q_EVEN[t]
                    qq = jnp.where(parity == 0, q_e, (q_e + 2) % 4)
                    blk = jnp.dot(
                        chord_ref[t], w, preferred_element_type=jnp.float32
                    )
                    out_ref[pl.ds(o * m_per + qq * q_rows, q_rows), :] = blk
                    amax = jnp.maximum(amax, jnp.max(jnp.abs(blk)))

        for t in range(8):
            chord_desc(0, 0, t).wait_send()

        amax_ref[p] = jnp.full((8, 128), amax, jnp.float32)
        send_descs = []
        for d in range(1, N_DEV):
            tgt = (p + d) % N_DEV
            s = pltpu.make_async_remote_copy(
                src_ref=amax_ref.at[p],
                dst_ref=amax_ref.at[p],
                send_sem=a_send_sems.at[d],
                recv_sem=a_recv_sems.at[d],
                device_id=(tgt,),
                device_id_type=pl.DeviceIdType.MESH,
            )
            s.start()
            send_descs.append(s)
        for d in range(1, N_DEV):
            src_pos = (p - d) % N_DEV
            r = pltpu.make_async_remote_copy(
                src_ref=amax_ref.at[p],
                dst_ref=amax_ref.at[src_pos],
                send_sem=a_send_sems.at[d],
                recv_sem=a_recv_sems.at[d],
                device_id=(src_pos,),
                device_id_type=pl.DeviceIdType.MESH,
            )
            r.wait_recv()
        for s in send_descs:
            s.wait_send()

        g_amax = jnp.max(amax_ref[...])
        inv = 127.0 / g_amax
        scale = g_amax * (1.0 / 127.0)
        for rblk in range(N_DEV):
            sl = pl.ds(rblk * m_per, m_per)
            y = out_ref[sl, :]
            qv = jnp.clip(jnp.round(y * inv), -127.0, 127.0)
            out_ref[sl, :] = qv * scale

    return pl.pallas_call(
        body,
        out_shape=jax.ShapeDtypeStruct((N_DEV * m_per, n_per), jnp.float32),
        in_specs=[
            pl.BlockSpec(memory_space=pltpu.VMEM),
            pl.BlockSpec(memory_space=pl.ANY),
        ],
        out_specs=pl.BlockSpec(memory_space=pltpu.VMEM),
        scratch_shapes=[
            pltpu.VMEM((k, n_per), jnp.bfloat16),
            pltpu.VMEM((k // 4, n_per), jnp.float32),
            pltpu.VMEM((N_FLOW, 2, q_rows, k), jnp.bfloat16),
            pltpu.VMEM((8, q_rows, k), jnp.bfloat16),
            pltpu.VMEM((N_DEV, 8, 128), jnp.float32),
            pltpu.SemaphoreType.DMA((N_FLOW, 2)),
            pltpu.SemaphoreType.DMA((N_FLOW, 2)),
            pltpu.SemaphoreType.DMA((8,)),
            pltpu.SemaphoreType.DMA((8,)),
            pltpu.SemaphoreType.DMA((N_DEV,)),
            pltpu.SemaphoreType.DMA((N_DEV,)),
            pltpu.SemaphoreType.DMA,
        ],
        compiler_params=pltpu.CompilerParams(
            collective_id=0, vmem_limit_bytes=100 * 1024 * 1024
        ),
    )(x, w_mat)


# device time: 155712 ns/iter; 2.6744x vs baseline; 1.0003x over previous
import jax
import jax.numpy as jnp
from jax import lax
from jax.experimental import pallas as pl
from jax.experimental.pallas import tpu as pltpu

N_DEV = 8
N_FLOW = 4
N_HOP = 5

RING = [0, 4, 7, 3, 2, 6, 5, 1]
RIDX = {p: i for i, p in enumerate(RING)}
CHORD = {
    p: RING[(RIDX[p] + (3 if RIDX[p] % 2 == 0 else -3)) % 8] for p in range(8)
}

FLOWPAIR_EVEN = [(2, 3), (2, 3), (0, 1), (0, 1)]
FLOWPAIR_ODD = [(0, 1), (0, 1), (2, 3), (2, 3)]
CHORD_ROFF_EVEN = [2, 2, 1, 1, -2, -2, -1, -1]
CHORD_Q_EVEN = [0, 1, 0, 1, 2, 3, 2, 3]


def _sel(p, table):
    v = jnp.int32(table[0])
    for j in range(1, 8):
        v = jnp.where(p == j, jnp.int32(table[j]), v)
    return v


def _ring_tab(off):
    return [RING[(RIDX[p] + off) % 8] for p in range(8)]


def kernel(x, w_mat):
    m_per, k = x.shape
    _, n_per = w_mat.shape
    q_rows = m_per // 4

    def body(
        x_ref,
        w_ref,
        out_ref,
        w_bf_ref,
        w_stage_ref,
        flow_ref,
        chord_ref,
        amax_ref,
        send_sems,
        recv_sems,
        c_send_sems,
        c_recv_sems,
        a_send_sems,
        a_recv_sems,
        w_dma_sem,
    ):
        p = lax.axis_index("i")
        right = _sel(p, _ring_tab(1))
        left = _sel(p, _ring_tab(-1))
        chordp = _sel(p, [CHORD[q] for q in range(8)])
        parity = _sel(p, [RIDX[q] % 2 for q in range(8)])

        kq = k // 4
        cp0 = pltpu.make_async_copy(
            w_ref.at[pl.ds(0, kq), :], w_stage_ref, w_dma_sem
        )
        cp0.start()

        barrier_sem = pltpu.get_barrier_semaphore()
        for nbr in [left, right, chordp]:
            pl.semaphore_signal(
                barrier_sem,
                inc=1,
                device_id=(nbr,),
                device_id_type=pl.DeviceIdType.MESH,
            )
        pl.semaphore_wait(barrier_sem, 3)

        flow_dst = [right, right, left, left]

        def make_rdma(f, h):
            ss, rs = h % 2, (h + 1) % 2
            return pltpu.make_async_remote_copy(
                src_ref=flow_ref.at[f, ss],
                dst_ref=flow_ref.at[f, rs],
                send_sem=send_sems.at[f, ss],
                recv_sem=recv_sems.at[f, rs],
                device_id=(flow_dst[f],),
                device_id_type=pl.DeviceIdType.MESH,
            )

        def chord_desc(src_f, src_slot, t):
            return pltpu.make_async_remote_copy(
                src_ref=flow_ref.at[src_f, src_slot],
                dst_ref=chord_ref.at[t],
                send_sem=c_send_sems.at[t],
                recv_sem=c_recv_sems.at[t],
                device_id=(chordp,),
                device_id_type=pl.DeviceIdType.MESH,
            )

        flow_order = (0, 2, 1, 3)

        rdmas = [None] * N_FLOW
        for f in flow_order:
            flow_ref[f, 0] = x_ref[pl.ds(f * q_rows, q_rows), :].astype(jnp.bfloat16)
            rdmas[f] = make_rdma(f, 0)
            rdmas[f].start()

        for i in range(4):
            cp = pltpu.make_async_copy(
                w_ref.at[pl.ds(i * kq, kq), :], w_stage_ref, w_dma_sem
            )
            if i > 0:
                cp.start()
            cp.wait()
            w_bf_ref[pl.ds(i * kq, kq), :] = w_stage_ref[...].astype(jnp.bfloat16)
        w = w_bf_ref[...]
        amax = jnp.float32(0.0)

        for h in range(N_HOP):
            rs = (h + 1) % 2
            for f in flow_order:
                rdmas[f].wait()
                if h < N_HOP - 1:
                    rdmas[f] = make_rdma(f, h + 1)
                    rdmas[f].start()

            if h < 4:

                @pl.when(parity == 0)
                def _():
                    for j, f in enumerate(FLOWPAIR_EVEN[h]):
                        chord_desc(f, rs, 2 * h + j).start()

                @pl.when(parity == 1)
                def _():
                    for j, f in enumerate(FLOWPAIR_ODD[h]):
                        chord_desc(f, rs, 2 * h + j).start()

            if h == 0:
                for f in range(N_FLOW):
                    xq = x_ref[pl.ds(f * q_rows, q_rows), :].astype(jnp.bfloat16)
                    blk = jnp.dot(xq, w, preferred_element_type=jnp.float32)
                    out_ref[pl.ds(p * m_per + f * q_rows, q_rows), :] = blk
                    amax = jnp.maximum(amax, jnp.max(jnp.abs(blk)))

            for f in range(N_FLOW):
                off = -(h + 1) if f < 2 else (h + 1)
                o = _sel(p, _ring_tab(off))
                blk = jnp.dot(flow_ref[f, rs], w, preferred_element_type=jnp.float32)
                out_ref[pl.ds(o * m_per + f * q_rows, q_rows), :] = blk
                amax = jnp.maximum(amax, jnp.max(jnp.abs(blk)))

            if h >= 1:
                for t in (2 * (h - 1), 2 * h - 1):
                    chord_desc(0, 0, t).wait_recv()
                    roff_e = CHORD_ROFF_EVEN[t]
                    o = jnp.where(
                        parity == 0,
                        _sel(p, _ring_tab(roff_e)),
                        _sel(p, _ring_tab(-roff_e)),
                    )
                    q_e = CHORD_Q_EVEN[t]
                    qq = jnp.where(parity == 0, q_e, (q_e + 2) % 4)
                    blk = jnp.dot(
                        chord_ref[t], w, preferred_element_type=jnp.float32
                    )
                    out_ref[pl.ds(o * m_per + qq * q_rows, q_rows), :] = blk
                    amax = jnp.maximum(amax, jnp.max(jnp.abs(blk)))

        amax_ref[p] = jnp.full((8, 128), amax, jnp.float32)
        send_descs = []
        for d in range(1, N_DEV):
            tgt = (p + d) % N_DEV
            s = pltpu.make_async_remote_copy(
                src_ref=amax_ref.at[p],
                dst_ref=amax_ref.at[p],
                send_sem=a_send_sems.at[d],
                recv_sem=a_recv_sems.at[d],
                device_id=(tgt,),
                device_id_type=pl.DeviceIdType.MESH,
            )
            s.start()
            send_descs.append(s)

        for t in range(8):
            chord_desc(0, 0, t).wait_send()
        for d in range(1, N_DEV):
            src_pos = (p - d) % N_DEV
            r = pltpu.make_async_remote_copy(
                src_ref=amax_ref.at[p],
                dst_ref=amax_ref.at[src_pos],
                send_sem=a_send_sems.at[d],
                recv_sem=a_recv_sems.at[d],
                device_id=(src_pos,),
                device_id_type=pl.DeviceIdType.MESH,
            )
            r.wait_recv()
        for s in send_descs:
            s.wait_send()

        g_amax = jnp.max(amax_ref[...])
        inv = 127.0 / g_amax
        scale = g_amax * (1.0 / 127.0)
        for rblk in range(N_DEV):
            sl = pl.ds(rblk * m_per, m_per)
            y = out_ref[sl, :]
            qv = jnp.clip(jnp.round(y * inv), -127.0, 127.0)
            out_ref[sl, :] = qv * scale

    return pl.pallas_call(
        body,
        out_shape=jax.ShapeDtypeStruct((N_DEV * m_per, n_per), jnp.float32),
        in_specs=[
            pl.BlockSpec(memory_space=pltpu.VMEM),
            pl.BlockSpec(memory_space=pl.ANY),
        ],
        out_specs=pl.BlockSpec(memory_space=pltpu.VMEM),
        scratch_shapes=[
            pltpu.VMEM((k, n_per), jnp.bfloat16),
            pltpu.VMEM((k // 4, n_per), jnp.float32),
            pltpu.VMEM((N_FLOW, 2, q_rows, k), jnp.bfloat16),
            pltpu.VMEM((8, q_rows, k), jnp.bfloat16),
            pltpu.VMEM((N_DEV, 8, 128), jnp.float32),
            pltpu.SemaphoreType.DMA((N_FLOW, 2)),
            pltpu.SemaphoreType.DMA((N_FLOW, 2)),
            pltpu.SemaphoreType.DMA((8,)),
            pltpu.SemaphoreType.DMA((8,)),
            pltpu.SemaphoreType.DMA((N_DEV,)),
            pltpu.SemaphoreType.DMA((N_DEV,)),
            pltpu.SemaphoreType.DMA,
        ],
        compiler_params=pltpu.CompilerParams(
            collective_id=0, vmem_limit_bytes=100 * 1024 * 1024
        ),
    )(x, w_mat)
